# baseline (speedup 1.0000x reference)
import functools

import jax
import jax.numpy as jnp
from jax import lax
from jax.experimental import pallas as pl
from jax.experimental.pallas import tpu as pltpu
from jax.experimental.pallas import tpu_sc as plsc

B = 4096
D = 4096
L = 16

NUM_CORES = 2
NUM_SUBCORES = 16
NW = NUM_CORES * NUM_SUBCORES
ROWS_PER_W = B // NW
CHUNK = 8
NCHUNKS = ROWS_PER_W // CHUNK
VECS_PER_ROW = D // L
UNROLL = 8


def _body(x_hbm, t_hbm, emb_hbm, out_hbm, idx_v, x_buf, emb_buf, sem_x, sem_e):
    wid = lax.axis_index("s") * NUM_CORES + lax.axis_index("c")
    base = wid * ROWS_PER_W

    pltpu.sync_copy(t_hbm.at[pl.ds(base, ROWS_PER_W)], idx_v)

    def chunk_step(c, carry):
        row0 = base + c * CHUNK
        cp_x = pltpu.async_copy(x_hbm.at[pl.ds(row0, CHUNK)], x_buf, sem_x)
        cp_e = pltpu.async_copy(
            emb_hbm.at[idx_v.at[pl.ds(c * CHUNK, CHUNK)]], emb_buf, sem_e
        )
        cp_x.wait()
        cp_e.wait()

        for r in range(CHUNK):
            def add_body(j, _):
                for u in range(UNROLL):
                    off = j * (UNROLL * L) + u * L
                    x_buf[r, pl.ds(off, L)] = (
                        x_buf[r, pl.ds(off, L)] + emb_buf[r, pl.ds(off, L)]
                    )
                return 0

            lax.fori_loop(0, VECS_PER_ROW // UNROLL, add_body, 0)

        pltpu.sync_copy(x_buf, out_hbm.at[pl.ds(row0, CHUNK)])
        return carry

    lax.fori_loop(0, NCHUNKS, chunk_step, 0)


def kernel(x, t, time_embeddings):
    t_flat = t.reshape(-1).astype(jnp.int32)
    mesh = plsc.VectorSubcoreMesh(core_axis_name="c", subcore_axis_name="s")
    run = pl.kernel(
        _body,
        mesh=mesh,
        out_type=jax.ShapeDtypeStruct((B, D), jnp.float32),
        scratch_types=[
            pltpu.VMEM((ROWS_PER_W,), jnp.int32),
            pltpu.VMEM((CHUNK, D), jnp.float32),
            pltpu.VMEM((CHUNK, D), jnp.float32),
            pltpu.SemaphoreType.DMA,
            pltpu.SemaphoreType.DMA,
        ],
    )
    return run(x, t_flat, time_embeddings)


# speedup vs baseline: 1.4177x; 1.4177x over previous
import jax
import jax.numpy as jnp
from jax import lax
from jax.experimental import pallas as pl
from jax.experimental.pallas import tpu as pltpu
from jax.experimental.pallas import tpu_sc as plsc

B = 4096
D = 4096
L = 16

NUM_CORES = 2
NUM_SUBCORES = 16
NW = NUM_CORES * NUM_SUBCORES
ROWS_PER_W = B // NW
CHUNK = 4
NCHUNKS = ROWS_PER_W // CHUNK
VECS_PER_ROW = D // L
UNROLL = 8


def _body(x_hbm, t_hbm, emb_hbm, out_hbm,
          idx_v, x0, x1, e0, e1,
          sx0, sx1, se0, se1, so0, so1):
    x_bufs = (x0, x1)
    e_bufs = (e0, e1)
    sem_x = (sx0, sx1)
    sem_e = (se0, se1)
    sem_o = (so0, so1)

    wid = lax.axis_index("s") * NUM_CORES + lax.axis_index("c")
    base = wid * ROWS_PER_W

    pltpu.sync_copy(t_hbm.at[wid], idx_v)

    def load(c, b):
        row0 = base + c * CHUNK
        pltpu.async_copy(x_hbm.at[pl.ds(row0, CHUNK)], x_bufs[b], sem_x[b])
        pltpu.async_copy(emb_hbm.at[idx_v.at[c]], e_bufs[b], sem_e[b])

    def wait_load(c, b):
        row0 = base + c * CHUNK
        pltpu.make_async_copy(
            x_hbm.at[pl.ds(row0, CHUNK)], x_bufs[b], sem_x[b]).wait()
        pltpu.make_async_copy(
            emb_hbm.at[idx_v.at[c]], e_bufs[b], sem_e[b]).wait()

    def store(c, b):
        row0 = base + c * CHUNK
        pltpu.async_copy(x_bufs[b], out_hbm.at[pl.ds(row0, CHUNK)], sem_o[b])

    def wait_store(c, b):
        row0 = base + c * CHUNK
        pltpu.make_async_copy(
            x_bufs[b], out_hbm.at[pl.ds(row0, CHUNK)], sem_o[b]).wait()

    load(0, 0)

    def pair_step(g, carry):
        for b in range(2):
            ob = 1 - b
            cc = 2 * g + b
            wait_load(cc, b)

            @pl.when(cc >= 1)
            def _():
                wait_store(cc - 1, ob)

            @pl.when(cc + 1 < NCHUNKS)
            def _():
                load(cc + 1, ob)

            for r in range(CHUNK):
                def add_body(j, _, r=r, b=b):
                    for u in range(UNROLL):
                        off = j * (UNROLL * L) + u * L
                        x_bufs[b][r, pl.ds(off, L)] = (
                            x_bufs[b][r, pl.ds(off, L)]
                            + e_bufs[b][r, pl.ds(off, L)]
                        )
                    return 0

                lax.fori_loop(0, VECS_PER_ROW // UNROLL, add_body, 0)

            store(cc, b)
        return carry

    lax.fori_loop(0, NCHUNKS // 2, pair_step, 0)
    wait_store(NCHUNKS - 1, (NCHUNKS - 1) % 2)


def kernel(x, t, time_embeddings):
    t_grid = t.reshape(NW, NCHUNKS, CHUNK).astype(jnp.int32)
    mesh = plsc.VectorSubcoreMesh(core_axis_name="c", subcore_axis_name="s")
    run = pl.kernel(
        _body,
        mesh=mesh,
        out_type=jax.ShapeDtypeStruct((B, D), jnp.float32),
        scratch_types=[
            pltpu.VMEM((NCHUNKS, CHUNK), jnp.int32),
            pltpu.VMEM((CHUNK, D), jnp.float32),
            pltpu.VMEM((CHUNK, D), jnp.float32),
            pltpu.VMEM((CHUNK, D), jnp.float32),
            pltpu.VMEM((CHUNK, D), jnp.float32),
            pltpu.SemaphoreType.DMA,
            pltpu.SemaphoreType.DMA,
            pltpu.SemaphoreType.DMA,
            pltpu.SemaphoreType.DMA,
            pltpu.SemaphoreType.DMA,
            pltpu.SemaphoreType.DMA,
        ],
    )
    return run(x, t_grid, time_embeddings)
